# baseline (speedup 1.0000x reference)
import functools

import jax
import jax.numpy as jnp
from jax import lax
from jax.experimental import pallas as pl
from jax.experimental.pallas import tpu as pltpu
from jax.experimental.pallas import tpu_sc as plsc

_HIDDEN = 128
_NCOLORS = 10
_KPAD = 16
_WIN = 128


def _build_tables(gt_ref, tab_ref, pos_ref, t_ref, idx_ref):
    h, w, b = gt_ref.shape
    pos2 = pos_ref[...].reshape(h * w, _HIDDEN)
    t_ref[...] = (pos2[:, None, :] + tab_ref[...][None, :, :]).reshape(
        h * w * _KPAD, _HIDDEN)
    ih = jax.lax.broadcasted_iota(jnp.int32, (h, w, b), 0)
    iw = jax.lax.broadcasted_iota(jnp.int32, (h, w, b), 1)
    idx_ref[...] = (ih * w + iw) * _KPAD + gt_ref[...]


def kernel(grid, color_embed, pos_embed):
    b, h, w = grid.shape
    n = b * h * w
    gt = jnp.transpose(grid.astype(jnp.int32), (1, 2, 0))
    tab = jnp.zeros((_KPAD, _HIDDEN), jnp.float32).at[:_NCOLORS].set(color_embed)
    t, idx = pl.pallas_call(
        _build_tables,
        grid=(1,),
        in_specs=[
            pl.BlockSpec((h, w, b), lambda i: (0, 0, 0)),
            pl.BlockSpec((_KPAD, _HIDDEN), lambda i: (0, 0)),
            pl.BlockSpec((h, w, _HIDDEN), lambda i: (0, 0, 0)),
        ],
        out_specs=[
            pl.BlockSpec((h * w * _KPAD, _HIDDEN), lambda i: (0, 0)),
            pl.BlockSpec((h, w, b), lambda i: (0, 0, 0)),
        ],
        out_shape=[
            jax.ShapeDtypeStruct((h * w * _KPAD, _HIDDEN), jnp.float32),
            jax.ShapeDtypeStruct((h, w, b), jnp.int32),
        ],
    )(gt, tab, pos_embed[:h, :w])

    mesh = plsc.VectorSubcoreMesh(core_axis_name="c", subcore_axis_name="s")

    @functools.partial(
        pl.kernel, mesh=mesh,
        out_type=jax.ShapeDtypeStruct((n, _HIDDEN), jnp.float32))
    def sc_gather(t_hbm, i_hbm, o_hbm):
        def body(i_vmem, o_vmem):
            pltpu.sync_copy(t_hbm.at[i_vmem.at[0]], o_vmem)

        pltpu.emit_pipeline(
            body,
            grid=(n // _WIN,),
            in_specs=[pl.BlockSpec((1, _WIN), index_map=lambda i: (0, i))],
            out_specs=[pl.BlockSpec((_WIN, _HIDDEN), index_map=lambda i: (i, 0))],
            core_axis_name=("c", "s"),
            dimension_semantics=(pltpu.PARALLEL,),
        )(i_hbm, o_hbm)

    out = sc_gather(t, idx.reshape(1, n))
    return jnp.transpose(out.reshape(h, w, b, _HIDDEN), (2, 0, 1, 3))


# speedup vs baseline: 5.3689x; 5.3689x over previous
import jax
import jax.numpy as jnp
from jax.experimental import pallas as pl
from jax.experimental.pallas import tpu as pltpu

_HIDDEN = 128
_NCOLORS = 10
_KPAD = 16
_LB = 1024
_HB = 1


def _embed_block(grid_ref, tab_ref, pos_ref, out_ref):
    hb, w, lb = grid_ref.shape
    g = grid_ref[...]
    oh = (g[..., None] == jax.lax.broadcasted_iota(
        jnp.int32, (hb, w, lb, _KPAD), 3)).astype(jnp.float32)
    x = jnp.dot(oh.reshape(hb * w * lb, _KPAD), tab_ref[...],
                preferred_element_type=jnp.float32)
    out_ref[...] = x.reshape(hb, w, lb, _HIDDEN) + pos_ref[...][:, :, None, :]


def kernel(grid, color_embed, pos_embed):
    b, h, w = grid.shape
    gt = jnp.transpose(grid.astype(jnp.int32), (1, 2, 0))
    tab = jnp.zeros((_KPAD, _HIDDEN), jnp.float32).at[:_NCOLORS].set(color_embed)
    pos = pos_embed[:h, :w]
    out = pl.pallas_call(
        _embed_block,
        grid=(h // _HB,),
        in_specs=[
            pl.BlockSpec((_HB, w, _LB), lambda i: (i, 0, 0)),
            pl.BlockSpec((_KPAD, _HIDDEN), lambda i: (0, 0)),
            pl.BlockSpec((_HB, w, _HIDDEN), lambda i: (i, 0, 0)),
        ],
        out_specs=pl.BlockSpec((_HB, w, _LB, _HIDDEN), lambda i: (i, 0, 0, 0)),
        out_shape=jax.ShapeDtypeStruct((h, w, b, _HIDDEN), jnp.float32),
        compiler_params=pltpu.CompilerParams(
            dimension_semantics=("parallel",)),
    )(gt, tab, pos)
    return jnp.transpose(out, (2, 0, 1, 3))
